# baseline (speedup 1.0000x reference)
import jax
import jax.numpy as jnp
from jax import lax
from jax.experimental import pallas as pl
from jax.experimental.pallas import tpu as pltpu
from jax.experimental.pallas import tpu_sc as plsc

DIM = 17
TSZ = DIM * DIM * DIM
TPAD = 4920
BINSIZE = 1.000001 / (DIM - 1)
INV_BIN = float(1.0 / BINSIZE)

NC, NS, L = 2, 16, 16
NW = NC * NS

H = W = 512
N_IMG = 16
PIX_PER_IMG = H * W
PIX_PER_W = N_IMG * PIX_PER_IMG // NW
CH = 8192
NCHUNK = PIX_PER_W // CH

_PAIR_OFFS = (0, DIM, DIM * DIM, DIM * DIM + DIM)


def _sc_body(lut_hbm, x_hbm, out_hbm,
             lutr, lutg, lutb,
             in0, in1, ou0, ou1,
             sem_i0, sem_i1, sem_o0, sem_o1):
    wid = lax.axis_index("s") * NC + lax.axis_index("c")
    img = wid // 2
    half = wid % 2
    row_base = half * (H // 2)

    inbufs = (in0, in1)
    obufs = (ou0, ou1)
    sem_in = (sem_i0, sem_i1)
    sem_out = (sem_o0, sem_o1)

    pltpu.sync_copy(lut_hbm.at[pl.ds(0 * TPAD, TPAD)], lutr)
    pltpu.sync_copy(lut_hbm.at[pl.ds(1 * TPAD, TPAD)], lutg)
    pltpu.sync_copy(lut_hbm.at[pl.ds(2 * TPAD, TPAD)], lutb)

    ROWS = CH // W

    def issue_in(k, slot):
        r0 = row_base + k * ROWS
        return [pltpu.async_copy(
            x_hbm.at[img, :, pl.ds(r0, ROWS)],
            inbufs[slot], sem_in[slot])]

    def issue_out(k, slot):
        r0 = row_base + k * ROWS
        return [pltpu.async_copy(
            obufs[slot],
            out_hbm.at[img, :, pl.ds(r0, ROWS)], sem_out[slot])]

    def compute(slot):
        ibuf = inbufs[slot]
        obuf = obufs[slot]

        @plsc.parallel_loop(0, CH, step=L, unroll=2)
        def vec_body(p):
            row = p >> 9
            s = pl.ds(p & (W - 1), L)
            tr = ibuf[0, row, s] * INV_BIN
            tg = ibuf[1, row, s] * INV_BIN
            tb = ibuf[2, row, s] * INV_BIN
            ir = tr.astype(jnp.int32)
            ig = tg.astype(jnp.int32)
            ib = tb.astype(jnp.int32)
            dr = tr - ir.astype(jnp.float32)
            dg = tg - ig.astype(jnp.float32)
            db = tb - ib.astype(jnp.float32)
            idx0 = ir + ig * DIM + ib * (DIM * DIM)

            r0 = 1.0 - dr
            g0 = 1.0 - dg
            b0 = 1.0 - db
            gb00 = g0 * b0
            gb10 = dg * b0
            gb01 = g0 * db
            gb11 = dg * db
            ws = (r0 * gb00, dr * gb00, r0 * gb10, dr * gb10,
                  r0 * gb01, dr * gb01, r0 * gb11, dr * gb11)
            idxs = tuple(idx0 + o for o in _PAIR_OFFS)

            for c, table in enumerate((lutr, lutg, lutb)):
                acc = None
                for j in range(4):
                    v = plsc.load_gather(table, [idxs[j]])
                    va = plsc.bitcast(v << 16, jnp.float32)
                    vb = plsc.bitcast(v & jnp.int32(-65536), jnp.float32)
                    term = ws[2 * j] * va + ws[2 * j + 1] * vb
                    acc = term if acc is None else acc + term
                obuf[c, row, s] = acc

    in_descs = [None, None]
    out_descs = [None, None]
    in_descs[0] = issue_in(0, 0)
    for k in range(NCHUNK):
        slot = k % 2
        if k + 1 < NCHUNK:
            in_descs[1 - slot] = issue_in(k + 1, 1 - slot)
        for d in in_descs[slot]:
            d.wait()
        if out_descs[slot] is not None:
            for d in out_descs[slot]:
                d.wait()
        compute(slot)
        out_descs[slot] = issue_out(k, slot)
    for descs in out_descs:
        for d in descs:
            d.wait()


@jax.jit
def _lut3d_sc(lut_pack_flat, x_flat):
    mesh = plsc.VectorSubcoreMesh(core_axis_name="c", subcore_axis_name="s",
                                  num_cores=NC, num_subcores=NS)
    run = pl.kernel(
        _sc_body,
        out_type=jax.ShapeDtypeStruct((N_IMG, 3, H, W), jnp.float32),
        mesh=mesh,
        compiler_params=pltpu.CompilerParams(needs_layout_passes=False),
        scratch_types=[
            pltpu.VMEM((TPAD,), jnp.int32),
            pltpu.VMEM((TPAD,), jnp.int32),
            pltpu.VMEM((TPAD,), jnp.int32),
        ] + [pltpu.VMEM((3, CH // W, W), jnp.float32)] * 4 + [
            pltpu.SemaphoreType.DMA,
            pltpu.SemaphoreType.DMA,
            pltpu.SemaphoreType.DMA,
            pltpu.SemaphoreType.DMA,
        ],
    )
    return run(lut_pack_flat, x_flat)


def _pack_lut(lut):
    lut3 = lut.reshape(3, TSZ)
    lo = lut3
    hi = jnp.concatenate([lut3[:, 1:], jnp.zeros((3, 1), lut3.dtype)], axis=1)
    lo_b = jax.lax.bitcast_convert_type(
        lo.astype(jnp.bfloat16), jnp.uint16).astype(jnp.uint32)
    hi_b = jax.lax.bitcast_convert_type(
        hi.astype(jnp.bfloat16), jnp.uint16).astype(jnp.uint32)
    packed = jax.lax.bitcast_convert_type((hi_b << 16) | lo_b, jnp.int32)
    return jnp.pad(packed, ((0, 0), (0, TPAD - TSZ)))


def kernel(lut, x):
    lut_pack = _pack_lut(lut)
    return _lut3d_sc(lut_pack.reshape(-1), x)


# speedup vs baseline: 1.1554x; 1.1554x over previous
import jax
import jax.numpy as jnp
from jax import lax
from jax.experimental import pallas as pl
from jax.experimental.pallas import tpu as pltpu
from jax.experimental.pallas import tpu_sc as plsc

DIM = 17
TSZ = DIM * DIM * DIM
TPAD = 4920
BINSIZE = 1.000001 / (DIM - 1)
INV_BIN = float(1.0 / BINSIZE)

NC, NS, L = 2, 16, 16
NW = NC * NS

H = W = 512
N_IMG = 16
PIX_PER_IMG = H * W
PIX_PER_W = N_IMG * PIX_PER_IMG // NW
CH = 8192
NCHUNK = PIX_PER_W // CH

_PAIR_OFFS = (0, DIM, DIM * DIM, DIM * DIM + DIM)


def _sc_body(lut_hbm, x_hbm, out_hbm,
             lutr, lutg, lutb,
             in0, in1, ou0, ou1,
             sem_i0, sem_i1, sem_o0, sem_o1):
    wid = lax.axis_index("s") * NC + lax.axis_index("c")
    img = wid // 2
    half = wid % 2
    row_base = half * (H // 2)

    inbufs = (in0, in1)
    obufs = (ou0, ou1)
    sem_in = (sem_i0, sem_i1)
    sem_out = (sem_o0, sem_o1)

    pltpu.sync_copy(lut_hbm.at[pl.ds(0 * TPAD, TPAD)], lutr)
    pltpu.sync_copy(lut_hbm.at[pl.ds(1 * TPAD, TPAD)], lutg)
    pltpu.sync_copy(lut_hbm.at[pl.ds(2 * TPAD, TPAD)], lutb)

    ROWS = CH // W

    def issue_in(k, slot):
        r0 = row_base + k * ROWS
        return [pltpu.async_copy(
            x_hbm.at[img, :, pl.ds(r0, ROWS)],
            inbufs[slot], sem_in[slot])]

    def issue_out(k, slot):
        r0 = row_base + k * ROWS
        return [pltpu.async_copy(
            obufs[slot],
            out_hbm.at[img, :, pl.ds(r0, ROWS)], sem_out[slot])]

    def compute(slot):
        ibuf = inbufs[slot]
        obuf = obufs[slot]

        @plsc.parallel_loop(0, CH, step=L, unroll=1)
        def vec_body(p):
            row = p >> 9
            s = pl.ds(p & (W - 1), L)
            tr = ibuf[0, row, s] * INV_BIN
            tg = ibuf[1, row, s] * INV_BIN
            tb = ibuf[2, row, s] * INV_BIN
            ir = tr.astype(jnp.int32)
            ig = tg.astype(jnp.int32)
            ib = tb.astype(jnp.int32)
            dr = tr - ir.astype(jnp.float32)
            dg = tg - ig.astype(jnp.float32)
            db = tb - ib.astype(jnp.float32)
            idx0 = ir + ig * DIM + ib * (DIM * DIM)

            r0 = 1.0 - dr
            g0 = 1.0 - dg
            b0 = 1.0 - db
            gb00 = g0 * b0
            gb10 = dg * b0
            gb01 = g0 * db
            gb11 = dg * db
            ws = (r0 * gb00, dr * gb00, r0 * gb10, dr * gb10,
                  r0 * gb01, dr * gb01, r0 * gb11, dr * gb11)
            idxs = tuple(idx0 + o for o in _PAIR_OFFS)

            for c, table in enumerate((lutr, lutg, lutb)):
                acc = None
                for j in range(4):
                    v = plsc.load_gather(table, [idxs[j]])
                    va = plsc.bitcast(v << 16, jnp.float32)
                    vb = plsc.bitcast(v & jnp.int32(-65536), jnp.float32)
                    term = ws[2 * j] * va + ws[2 * j + 1] * vb
                    acc = term if acc is None else acc + term
                obuf[c, row, s] = acc

    in_descs = [None, None]
    out_descs = [None, None]
    in_descs[0] = issue_in(0, 0)
    for k in range(NCHUNK):
        slot = k % 2
        if k + 1 < NCHUNK:
            in_descs[1 - slot] = issue_in(k + 1, 1 - slot)
        for d in in_descs[slot]:
            d.wait()
        if out_descs[slot] is not None:
            for d in out_descs[slot]:
                d.wait()
        compute(slot)
        out_descs[slot] = issue_out(k, slot)
    for descs in out_descs:
        for d in descs:
            d.wait()


@jax.jit
def _lut3d_sc(lut_pack_flat, x_flat):
    mesh = plsc.VectorSubcoreMesh(core_axis_name="c", subcore_axis_name="s",
                                  num_cores=NC, num_subcores=NS)
    run = pl.kernel(
        _sc_body,
        out_type=jax.ShapeDtypeStruct((N_IMG, 3, H, W), jnp.float32),
        mesh=mesh,
        compiler_params=pltpu.CompilerParams(needs_layout_passes=False),
        scratch_types=[
            pltpu.VMEM((TPAD,), jnp.int32),
            pltpu.VMEM((TPAD,), jnp.int32),
            pltpu.VMEM((TPAD,), jnp.int32),
        ] + [pltpu.VMEM((3, CH // W, W), jnp.float32)] * 4 + [
            pltpu.SemaphoreType.DMA,
            pltpu.SemaphoreType.DMA,
            pltpu.SemaphoreType.DMA,
            pltpu.SemaphoreType.DMA,
        ],
    )
    return run(lut_pack_flat, x_flat)


def _pack_lut(lut):
    lut3 = lut.reshape(3, TSZ)
    lo = lut3
    hi = jnp.concatenate([lut3[:, 1:], jnp.zeros((3, 1), lut3.dtype)], axis=1)
    lo_b = jax.lax.bitcast_convert_type(
        lo.astype(jnp.bfloat16), jnp.uint16).astype(jnp.uint32)
    hi_b = jax.lax.bitcast_convert_type(
        hi.astype(jnp.bfloat16), jnp.uint16).astype(jnp.uint32)
    packed = jax.lax.bitcast_convert_type((hi_b << 16) | lo_b, jnp.int32)
    return jnp.pad(packed, ((0, 0), (0, TPAD - TSZ)))


def kernel(lut, x):
    lut_pack = _pack_lut(lut)
    return _lut3d_sc(lut_pack.reshape(-1), x)
